# Initial kernel scaffold; baseline (speedup 1.0000x reference)
import jax
import jax.numpy as jnp
from jax.experimental import pallas as pl


def kernel(x, table, ln_weight, ln_bias):
    raise NotImplementedError("write your pallas kernel here")


# speedup vs baseline: 1.8192x; 1.8192x over previous
import functools

import jax
import jax.numpy as jnp
from jax import lax
from jax.experimental import pallas as pl
from jax.experimental.pallas import tpu as pltpu
from jax.experimental.pallas import tpu_sc as plsc

D = 64
L = 16
NC = 2
NS = 16
NW = NC * NS
CHUNK = 1024
IDXW = 128
EPS = 1e-12


def _hsum(v):
    for k in (8, 4, 2, 1):
        idx = lax.iota(jnp.int32, L) ^ k
        v = v + v.at[idx].get(mode="promise_in_bounds")
    return v


def _rsqrt(x):
    i = lax.bitcast_convert_type(x, jnp.int32)
    i = jnp.int32(0x5F3759DF) - (i >> 1)
    y = lax.bitcast_convert_type(i, jnp.float32)
    for _ in range(3):
        y = y * (1.5 - (0.5 * x) * y * y)
    return y


@functools.partial(jax.jit, static_argnames=("n",))
def _sc_fused(table, x2d, w, b, *, n):
    per_w = n // NW
    n_chunks = per_w // CHUNK
    mesh = plsc.VectorSubcoreMesh(core_axis_name="c", subcore_axis_name="s")

    @functools.partial(
        pl.kernel,
        mesh=mesh,
        compiler_params=pltpu.CompilerParams(use_tc_tiling_on_sc=False),
        out_type=jax.ShapeDtypeStruct((n, D), jnp.float32),
        scratch_types=[
            pltpu.VMEM((CHUNK // IDXW, IDXW), jnp.int32),
            pltpu.VMEM((CHUNK, D), jnp.float32),
            pltpu.VMEM((D,), jnp.float32),
            pltpu.VMEM((D,), jnp.float32),
            pltpu.SemaphoreType.DMA,
        ],
    )
    def k(table_hbm, x2d_hbm, w_hbm, b_hbm, out_hbm, idx_v, rows_v, w_v, b_v, sem):
        wid = lax.axis_index("s") * NC + lax.axis_index("c")
        pltpu.sync_copy(w_hbm, w_v)
        pltpu.sync_copy(b_hbm, b_v)
        wv = [w_v[pl.ds(L * j, L)] for j in range(D // L)]
        bv = [b_v[pl.ds(L * j, L)] for j in range(D // L)]

        def chunk_body(g, carry):
            base = wid * per_w + g * CHUNK
            row0 = pl.multiple_of(base // IDXW, 8)
            pltpu.sync_copy(x2d_hbm.at[pl.ds(row0, CHUNK // IDXW)], idx_v)
            cps = [
                pltpu.async_copy(
                    table_hbm.at[idx_v.at[j]],
                    rows_v.at[pl.ds(j * IDXW, IDXW)],
                    sem,
                )
                for j in range(CHUNK // IDXW)
            ]
            for c in cps:
                c.wait()

            def row_body(r, carry2):
                v = [rows_v[r, pl.ds(L * j, L)] for j in range(D // L)]
                s = (v[0] + v[1]) + (v[2] + v[3])
                q = (v[0] * v[0] + v[1] * v[1]) + (v[2] * v[2] + v[3] * v[3])
                s = _hsum(s)
                q = _hsum(q)
                mu = s * (1.0 / D)
                var = q * (1.0 / D) - mu * mu
                rs = _rsqrt(jnp.maximum(var, 0.0) + EPS)
                for j in range(D // L):
                    rows_v[r, pl.ds(L * j, L)] = (v[j] - mu) * rs * wv[j] + bv[j]
                return carry2

            lax.fori_loop(0, CHUNK, row_body, 0)
            pltpu.sync_copy(rows_v, out_hbm.at[pl.ds(base, CHUNK)])
            return carry

        lax.fori_loop(0, n_chunks, chunk_body, 0)

    return k(table, x2d, w, b)


def kernel(x, table, ln_weight, ln_bias):
    batch, seq = x.shape
    n = batch * seq
    x2d = x.reshape(n // IDXW, IDXW).astype(jnp.int32)
    out = _sc_fused(table, x2d, ln_weight, ln_bias, n=n)
    return out.reshape(batch, seq, D)


# speedup vs baseline: 1.9429x; 1.0680x over previous
import functools

import jax
import jax.numpy as jnp
from jax import lax
from jax.experimental import pallas as pl
from jax.experimental.pallas import tpu as pltpu
from jax.experimental.pallas import tpu_sc as plsc

D = 64
L = 16
NC = 2
NS = 16
NW = NC * NS
CHUNK = 1024
IDXW = 128
EPS = 1e-12


def _rsqrt(x):
    i = lax.bitcast_convert_type(x, jnp.int32)
    i = jnp.int32(0x5F3759DF) - (i >> 1)
    y = lax.bitcast_convert_type(i, jnp.float32)
    for _ in range(3):
        y = y * (1.5 - (0.5 * x) * y * y)
    return y


@functools.partial(jax.jit, static_argnames=("n",))
def _sc_fused(table, x2d, w, b, *, n):
    per_w = n // NW
    n_chunks = per_w // CHUNK
    mesh = plsc.VectorSubcoreMesh(core_axis_name="c", subcore_axis_name="s")

    @functools.partial(
        pl.kernel,
        mesh=mesh,
        compiler_params=pltpu.CompilerParams(
            use_tc_tiling_on_sc=False, needs_layout_passes=False
        ),
        out_type=jax.ShapeDtypeStruct((n, D), jnp.float32),
        scratch_types=[
            pltpu.VMEM((CHUNK // IDXW, IDXW), jnp.int32),
            pltpu.VMEM((CHUNK, D), jnp.float32),
            pltpu.VMEM((D,), jnp.float32),
            pltpu.VMEM((D,), jnp.float32),
            pltpu.SemaphoreType.DMA,
        ],
    )
    def k(table_hbm, x2d_hbm, w_hbm, b_hbm, out_hbm, idx_v, rows_v, w_v, b_v, sem):
        wid = lax.axis_index("s") * NC + lax.axis_index("c")
        pltpu.sync_copy(w_hbm, w_v)
        pltpu.sync_copy(b_hbm, b_v)
        wv = [w_v[pl.ds(L * j, L)] for j in range(D // L)]
        bv = [b_v[pl.ds(L * j, L)] for j in range(D // L)]

        def chunk_body(g, carry):
            base = wid * per_w + g * CHUNK
            row0 = pl.multiple_of(base // IDXW, 8)
            pltpu.sync_copy(x2d_hbm.at[pl.ds(row0, CHUNK // IDXW)], idx_v)
            cps = [
                pltpu.async_copy(
                    table_hbm.at[idx_v.at[j]],
                    rows_v.at[pl.ds(j * IDXW, IDXW)],
                    sem,
                )
                for j in range(CHUNK // IDXW)
            ]
            for c in cps:
                c.wait()

            def group_body(g16, carry2):
                rvec = g16 * L + lax.iota(jnp.int32, L)
                cols = [jnp.full((L,), j, jnp.int32) for j in range(D)]
                s = jnp.zeros((L,), jnp.float32)
                q = jnp.zeros((L,), jnp.float32)
                for j in range(D):
                    t = plsc.load_gather(rows_v, [rvec, cols[j]])
                    s = s + t
                    q = q + t * t
                mu = s * (1.0 / D)
                var = q * (1.0 / D) - mu * mu
                rs = _rsqrt(jnp.maximum(var, 0.0) + EPS)
                r0 = g16 * L
                for r in range(L):
                    mu_r = mu[r]
                    a_r = rs[r]
                    v = [rows_v[r0 + r, pl.ds(L * jj, L)] for jj in range(D // L)]
                    for jj in range(D // L):
                        rows_v[r0 + r, pl.ds(L * jj, L)] = (
                            (v[jj] - mu_r) * a_r * wv[jj] + bv[jj]
                        )
                return carry2

            lax.fori_loop(0, CHUNK // L, group_body, 0)
            pltpu.sync_copy(rows_v, out_hbm.at[pl.ds(base, CHUNK)])
            return carry

        lax.fori_loop(0, n_chunks, chunk_body, 0)

    return k(table, x2d, w, b)


def kernel(x, table, ln_weight, ln_bias):
    batch, seq = x.shape
    n = batch * seq
    x2d = x.reshape(n // IDXW, IDXW).astype(jnp.int32)
    out = _sc_fused(table, x2d, ln_weight, ln_bias, n=n)
    return out.reshape(batch, seq, D)
